# baseline (speedup 1.0000x reference)
import dataclasses
import functools

import jax
import jax.numpy as jnp
from jax import lax
from jax.experimental import pallas as pl
from jax.experimental.pallas import tpu as pltpu
from jax.experimental.pallas import tpu_sc as plsc

D = 33
D2 = D * D
D3 = D * D * D
DPAD = 35944
L = 16
BLK = 1024

B, H, W = 16, 512, 512
NPIX = H * W


def _pipeline_body(lut0, lut1, lut2, in_ref, out_ref):
    @plsc.parallel_loop(0, BLK, step=L, unroll=4)
    def _(c):
        r = in_ref[0, 0, pl.ds(c, L)]
        g = in_ref[0, 1, pl.ds(c, L)]
        b = in_ref[0, 2, pl.ds(c, L)]

        x = jnp.clip(r * jnp.float32(D - 1), 0.0, jnp.float32(D - 1))
        y = jnp.clip(g * jnp.float32(D - 1), 0.0, jnp.float32(D - 1))
        z = jnp.clip(b * jnp.float32(D - 1), 0.0, jnp.float32(D - 1))

        xi = jnp.minimum(x.astype(jnp.int32), D - 2)
        yi = jnp.minimum(y.astype(jnp.int32), D - 2)
        zi = jnp.minimum(z.astype(jnp.int32), D - 2)
        wx = x - xi.astype(jnp.float32)
        wy = y - yi.astype(jnp.float32)
        wz = z - zi.astype(jnp.float32)
        ux = 1.0 - wx
        uy = 1.0 - wy
        uz = 1.0 - wz

        b00 = zi * D2 + yi * D
        b01 = b00 + D
        b10 = b00 + D2
        b11 = b10 + D
        i000 = b00 + xi
        i001 = i000 + 1
        i010 = b01 + xi
        i011 = i010 + 1
        i100 = b10 + xi
        i101 = i100 + 1
        i110 = b11 + xi
        i111 = i110 + 1

        wzy00 = uz * uy
        wzy01 = uz * wy
        wzy10 = wz * uy
        wzy11 = wz * wy
        w000 = wzy00 * ux
        w001 = wzy00 * wx
        w010 = wzy01 * ux
        w011 = wzy01 * wx
        w100 = wzy10 * ux
        w101 = wzy10 * wx
        w110 = wzy11 * ux
        w111 = wzy11 * wx

        for ch, lut in ((0, lut0), (1, lut1), (2, lut2)):
            acc = plsc.load_gather(lut, [i000]) * w000
            acc = acc + plsc.load_gather(lut, [i001]) * w001
            acc = acc + plsc.load_gather(lut, [i010]) * w010
            acc = acc + plsc.load_gather(lut, [i011]) * w011
            acc = acc + plsc.load_gather(lut, [i100]) * w100
            acc = acc + plsc.load_gather(lut, [i101]) * w101
            acc = acc + plsc.load_gather(lut, [i110]) * w110
            acc = acc + plsc.load_gather(lut, [i111]) * w111
            out_ref[0, ch, pl.ds(c, L)] = acc


def _lut3d_sc(img3, lutp):
    mesh = plsc.VectorSubcoreMesh(core_axis_name="c", subcore_axis_name="s")

    cp = pltpu.CompilerParams()
    if "needs_layout_passes" in pltpu.CompilerParams.__dataclass_fields__:
        cp = dataclasses.replace(cp, needs_layout_passes=False)

    @functools.partial(
        pl.kernel,
        compiler_params=cp,
        out_type=jax.ShapeDtypeStruct((B, 3, NPIX), jnp.float32),
        mesh=mesh,
        scratch_types=[
            pltpu.VMEM((DPAD,), jnp.float32),
            pltpu.VMEM((DPAD,), jnp.float32),
            pltpu.VMEM((DPAD,), jnp.float32),
        ],
    )
    def k(img_hbm, lut_hbm, out_hbm, lut0, lut1, lut2):
        pltpu.sync_copy(lut_hbm.at[pl.ds(0, DPAD)], lut0)
        pltpu.sync_copy(lut_hbm.at[pl.ds(DPAD, DPAD)], lut1)
        pltpu.sync_copy(lut_hbm.at[pl.ds(2 * DPAD, DPAD)], lut2)

        body = functools.partial(_pipeline_body, lut0, lut1, lut2)
        pltpu.emit_pipeline(
            body,
            grid=(B, NPIX // BLK),
            in_specs=[
                pl.BlockSpec((1, 3, BLK), index_map=lambda i, j: (i, 0, j)),
            ],
            out_specs=[
                pl.BlockSpec((1, 3, BLK), index_map=lambda i, j: (i, 0, j)),
            ],
            core_axis_name=("c", "s"),
            dimension_semantics=(pltpu.PARALLEL, pltpu.PARALLEL),
        )(img_hbm, out_hbm)

    return k(img3, lutp)


def kernel(img, LUT):
    img3 = img.reshape(B, 3, NPIX)
    lutp = jnp.pad(LUT.reshape(3, D3), ((0, 0), (0, DPAD - D3))).reshape(-1)
    out = _lut3d_sc(img3, lutp)
    return out.reshape(B, 3, H, W)


# speedup vs baseline: 1.5465x; 1.5465x over previous
import dataclasses
import functools

import jax
import jax.numpy as jnp
from jax import lax
from jax.experimental import pallas as pl
from jax.experimental.pallas import tpu as pltpu
from jax.experimental.pallas import tpu_sc as plsc

D = 33
D2 = D * D
D3 = D * D * D
DPAD = 35944
L = 16
BLK = 1024

B, H, W = 16, 512, 512
NPIX = H * W


def _pipeline_body(lut0, lut1, lut2, in_ref, out_ref):
    @plsc.parallel_loop(0, BLK, step=L, unroll=2)
    def _(c):
        r = in_ref[0, 0, pl.ds(c, L)]
        g = in_ref[0, 1, pl.ds(c, L)]
        b = in_ref[0, 2, pl.ds(c, L)]

        x = jnp.clip(r * jnp.float32(D - 1), 0.0, jnp.float32(D - 1))
        y = jnp.clip(g * jnp.float32(D - 1), 0.0, jnp.float32(D - 1))
        z = jnp.clip(b * jnp.float32(D - 1), 0.0, jnp.float32(D - 1))

        xi = jnp.minimum(x.astype(jnp.int32), D - 2)
        yi = jnp.minimum(y.astype(jnp.int32), D - 2)
        zi = jnp.minimum(z.astype(jnp.int32), D - 2)
        wx = x - xi.astype(jnp.float32)
        wy = y - yi.astype(jnp.float32)
        wz = z - zi.astype(jnp.float32)
        ux = 1.0 - wx
        uy = 1.0 - wy
        uz = 1.0 - wz

        b00 = zi * D2 + yi * D
        b01 = b00 + D
        b10 = b00 + D2
        b11 = b10 + D
        i000 = b00 + xi
        i001 = i000 + 1
        i010 = b01 + xi
        i011 = i010 + 1
        i100 = b10 + xi
        i101 = i100 + 1
        i110 = b11 + xi
        i111 = i110 + 1

        wzy00 = uz * uy
        wzy01 = uz * wy
        wzy10 = wz * uy
        wzy11 = wz * wy
        w000 = wzy00 * ux
        w001 = wzy00 * wx
        w010 = wzy01 * ux
        w011 = wzy01 * wx
        w100 = wzy10 * ux
        w101 = wzy10 * wx
        w110 = wzy11 * ux
        w111 = wzy11 * wx

        for ch, lut in ((0, lut0), (1, lut1), (2, lut2)):
            acc = plsc.load_gather(lut, [i000]) * w000
            acc = acc + plsc.load_gather(lut, [i001]) * w001
            acc = acc + plsc.load_gather(lut, [i010]) * w010
            acc = acc + plsc.load_gather(lut, [i011]) * w011
            acc = acc + plsc.load_gather(lut, [i100]) * w100
            acc = acc + plsc.load_gather(lut, [i101]) * w101
            acc = acc + plsc.load_gather(lut, [i110]) * w110
            acc = acc + plsc.load_gather(lut, [i111]) * w111
            out_ref[0, ch, pl.ds(c, L)] = acc


def _lut3d_sc(img3, lutp):
    mesh = plsc.VectorSubcoreMesh(core_axis_name="c", subcore_axis_name="s")

    cp = pltpu.CompilerParams()
    if "needs_layout_passes" in pltpu.CompilerParams.__dataclass_fields__:
        cp = dataclasses.replace(cp, needs_layout_passes=False)

    @functools.partial(
        pl.kernel,
        compiler_params=cp,
        out_type=jax.ShapeDtypeStruct((B, 3, NPIX), jnp.float32),
        mesh=mesh,
        scratch_types=[
            pltpu.VMEM((DPAD,), jnp.float32),
            pltpu.VMEM((DPAD,), jnp.float32),
            pltpu.VMEM((DPAD,), jnp.float32),
        ],
    )
    def k(img_hbm, lut_hbm, out_hbm, lut0, lut1, lut2):
        pltpu.sync_copy(lut_hbm.at[pl.ds(0, DPAD)], lut0)
        pltpu.sync_copy(lut_hbm.at[pl.ds(DPAD, DPAD)], lut1)
        pltpu.sync_copy(lut_hbm.at[pl.ds(2 * DPAD, DPAD)], lut2)

        body = functools.partial(_pipeline_body, lut0, lut1, lut2)
        pltpu.emit_pipeline(
            body,
            grid=(B, NPIX // BLK),
            in_specs=[
                pl.BlockSpec((1, 3, BLK), index_map=lambda i, j: (i, 0, j)),
            ],
            out_specs=[
                pl.BlockSpec((1, 3, BLK), index_map=lambda i, j: (i, 0, j)),
            ],
            core_axis_name=("c", "s"),
            dimension_semantics=(pltpu.PARALLEL, pltpu.PARALLEL),
        )(img_hbm, out_hbm)

    return k(img3, lutp)


def kernel(img, LUT):
    img3 = img.reshape(B, 3, NPIX)
    lutp = jnp.pad(LUT.reshape(3, D3), ((0, 0), (0, DPAD - D3))).reshape(-1)
    out = _lut3d_sc(img3, lutp)
    return out.reshape(B, 3, H, W)


# speedup vs baseline: 2.5062x; 1.6205x over previous
import dataclasses
import functools

import jax
import jax.numpy as jnp
from jax import lax
from jax.experimental import pallas as pl
from jax.experimental.pallas import tpu as pltpu
from jax.experimental.pallas import tpu_sc as plsc

D = 33
D2 = D * D
D3 = D * D * D
DPAD = 35944
L = 16
HBLK = 2

B, H, W = 16, 512, 512


def _row_loop(lut0, lut1, lut2, in_ref, out_ref, h):
    @plsc.parallel_loop(0, W, step=L, unroll=2)
    def _(c):
        r = in_ref[0, 0, h, pl.ds(c, L)]
        g = in_ref[0, 1, h, pl.ds(c, L)]
        b = in_ref[0, 2, h, pl.ds(c, L)]

        x = jnp.clip(r * jnp.float32(D - 1), 0.0, jnp.float32(D - 1))
        y = jnp.clip(g * jnp.float32(D - 1), 0.0, jnp.float32(D - 1))
        z = jnp.clip(b * jnp.float32(D - 1), 0.0, jnp.float32(D - 1))

        xi = jnp.minimum(x.astype(jnp.int32), D - 2)
        yi = jnp.minimum(y.astype(jnp.int32), D - 2)
        zi = jnp.minimum(z.astype(jnp.int32), D - 2)
        wx = x - xi.astype(jnp.float32)
        wy = y - yi.astype(jnp.float32)
        wz = z - zi.astype(jnp.float32)
        ux = 1.0 - wx
        uy = 1.0 - wy
        uz = 1.0 - wz

        b00 = zi * D2 + yi * D
        b01 = b00 + D
        b10 = b00 + D2
        b11 = b10 + D
        i000 = b00 + xi
        i001 = i000 + 1
        i010 = b01 + xi
        i011 = i010 + 1
        i100 = b10 + xi
        i101 = i100 + 1
        i110 = b11 + xi
        i111 = i110 + 1

        wzy00 = uz * uy
        wzy01 = uz * wy
        wzy10 = wz * uy
        wzy11 = wz * wy
        w000 = wzy00 * ux
        w001 = wzy00 * wx
        w010 = wzy01 * ux
        w011 = wzy01 * wx
        w100 = wzy10 * ux
        w101 = wzy10 * wx
        w110 = wzy11 * ux
        w111 = wzy11 * wx

        for ch, lut in ((0, lut0), (1, lut1), (2, lut2)):
            acc = plsc.load_gather(lut, [i000]) * w000
            acc = acc + plsc.load_gather(lut, [i001]) * w001
            acc = acc + plsc.load_gather(lut, [i010]) * w010
            acc = acc + plsc.load_gather(lut, [i011]) * w011
            acc = acc + plsc.load_gather(lut, [i100]) * w100
            acc = acc + plsc.load_gather(lut, [i101]) * w101
            acc = acc + plsc.load_gather(lut, [i110]) * w110
            acc = acc + plsc.load_gather(lut, [i111]) * w111
            out_ref[0, ch, h, pl.ds(c, L)] = acc


def _pipeline_body(lut0, lut1, lut2, in_ref, out_ref):
    for h in range(HBLK):
        _row_loop(lut0, lut1, lut2, in_ref, out_ref, h)


def _lut3d_sc(img, lutp):
    mesh = plsc.VectorSubcoreMesh(core_axis_name="c", subcore_axis_name="s")

    cp = pltpu.CompilerParams()
    if "needs_layout_passes" in pltpu.CompilerParams.__dataclass_fields__:
        cp = dataclasses.replace(cp, needs_layout_passes=False)

    @functools.partial(
        pl.kernel,
        compiler_params=cp,
        out_type=jax.ShapeDtypeStruct((B, 3, H, W), jnp.float32),
        mesh=mesh,
        scratch_types=[
            pltpu.VMEM((DPAD,), jnp.float32),
            pltpu.VMEM((DPAD,), jnp.float32),
            pltpu.VMEM((DPAD,), jnp.float32),
        ],
    )
    def k(img_hbm, lut_hbm, out_hbm, lut0, lut1, lut2):
        pltpu.sync_copy(lut_hbm.at[pl.ds(0, DPAD)], lut0)
        pltpu.sync_copy(lut_hbm.at[pl.ds(DPAD, DPAD)], lut1)
        pltpu.sync_copy(lut_hbm.at[pl.ds(2 * DPAD, DPAD)], lut2)

        body = functools.partial(_pipeline_body, lut0, lut1, lut2)
        pltpu.emit_pipeline(
            body,
            grid=(B, H // HBLK),
            in_specs=[
                pl.BlockSpec((1, 3, HBLK, W), index_map=lambda i, j: (i, 0, j, 0)),
            ],
            out_specs=[
                pl.BlockSpec((1, 3, HBLK, W), index_map=lambda i, j: (i, 0, j, 0)),
            ],
            core_axis_name=("c", "s"),
            dimension_semantics=(pltpu.PARALLEL, pltpu.PARALLEL),
        )(img_hbm, out_hbm)

    return k(img, lutp)


def kernel(img, LUT):
    lutp = jnp.pad(LUT.reshape(3, D3), ((0, 0), (0, DPAD - D3))).reshape(-1)
    return _lut3d_sc(img, lutp)
